# baseline (speedup 1.0000x reference)
import functools

import jax
import jax.numpy as jnp
from jax import lax
from jax.experimental import pallas as pl
from jax.experimental.pallas import tpu as pltpu
from jax.experimental.pallas import tpu_sc as plsc

N_NODES = 10000
N_HEDGES = 10000
NNZ = 320000
PAD = 10240
NW = 32
CHUNK = 128
NCH = 80
NCH_H = NCH // 2
REAL_PER_TILE = NNZ // NW
DUM_PER_TILE = NCH * CHUNK - REAL_PER_TILE
ROWS_PER_SUB = PAD // 16


def _make_segsum(F):
    mesh = plsc.VectorSubcoreMesh(core_axis_name="c", subcore_axis_name="s")

    @functools.partial(
        pl.kernel,
        mesh=mesh,
        out_type=jax.ShapeDtypeStruct((2, PAD, F), jnp.float32),
        scratch_types=[
            pltpu.VMEM((NCH_H, CHUNK), jnp.int32),
            pltpu.VMEM((NCH_H, CHUNK), jnp.int32),
            pltpu.VMEM((CHUNK, F), jnp.float32),
            pltpu.VMEM((CHUNK, F), jnp.float32),
            pltpu.VMEM_SHARED((PAD, F), jnp.float32),
            [pltpu.SemaphoreType.DMA] * 2,
        ],
    )
    def seg(table_hbm, src_hbm, dst_hbm, out_hbm, sidx_v, didx_v, rows0_v, rows1_v,
            acc_sh, gsems):
        c = lax.axis_index("c")
        s = lax.axis_index("s")
        wid = s * 2 + c
        bufs = (rows0_v, rows1_v)

        zvec = jnp.zeros((16,), jnp.float32)

        def _zrow(i, carry):
            for j in range(F // 16):
                rows0_v[i, pl.ds(j * 16, 16)] = zvec
            return carry

        lax.fori_loop(0, CHUNK, _zrow, 0)
        for k in range(ROWS_PER_SUB // CHUNK):
            pltpu.sync_copy(rows0_v, acc_sh.at[pl.ds(s * ROWS_PER_SUB + k * CHUNK, CHUNK)])
        plsc.subcore_barrier()

        def g_desc(j, t):
            return pltpu.make_async_copy(table_hbm.at[sidx_v.at[j]], bufs[t], gsems[t])

        for half in range(2):
            pltpu.sync_copy(src_hbm.at[wid, half], sidx_v)
            pltpu.sync_copy(dst_hbm.at[wid, half], didx_v)

            pltpu.async_copy(table_hbm.at[sidx_v.at[0]], bufs[0], gsems[0])

            def _pair(i, carry):
                for t in range(2):
                    j = i * 2 + t
                    g_desc(j, t).wait()
                    nxt = j + 1

                    @pl.when(nxt < NCH_H)
                    def _():
                        pltpu.async_copy(table_hbm.at[sidx_v.at[nxt]],
                                         bufs[1 - t], gsems[1 - t])

                    pltpu.sync_copy(bufs[t], acc_sh.at[didx_v.at[j]], add=True)
                return carry

            lax.fori_loop(0, NCH_H // 2, _pair, 0)

        plsc.subcore_barrier()
        pltpu.sync_copy(
            acc_sh.at[pl.ds(s * ROWS_PER_SUB, ROWS_PER_SUB)],
            out_hbm.at[c, pl.ds(s * ROWS_PER_SUB, ROWS_PER_SUB)],
        )

    return seg


def _tc_first(x, w, b, scale_out):
    n = x.shape[0]
    br = 1000

    def body(x_ref, w_ref, b_ref, so_ref, o_ref):
        h = jnp.dot(x_ref[...], w_ref[...], preferred_element_type=jnp.float32)
        o_ref[...] = (h + b_ref[...]) * so_ref[...]

    return pl.pallas_call(
        body,
        grid=(n // br,),
        in_specs=[
            pl.BlockSpec((br, 128), lambda i: (i, 0)),
            pl.BlockSpec((128, 128), lambda i: (0, 0)),
            pl.BlockSpec((1, 128), lambda i: (0, 0)),
            pl.BlockSpec((br, 1), lambda i: (i, 0)),
        ],
        out_specs=pl.BlockSpec((br, 128), lambda i: (i, 0)),
        out_shape=jax.ShapeDtypeStruct((n, 128), jnp.float32),
    )(x, w, b.reshape(1, -1), scale_out.reshape(-1, 1))


def _tc_mid(parts, scale_in, w, b, scale_out):
    br = 1024
    k = parts.shape[2]
    ko = w.shape[1]

    def body(p_ref, si_ref, w_ref, b_ref, so_ref, o_ref):
        a = p_ref[0] + p_ref[1]
        t = jnp.maximum(a * si_ref[...], 0.0)
        h = jnp.dot(t, w_ref[...], preferred_element_type=jnp.float32)
        o_ref[...] = (h + b_ref[...]) * so_ref[...]

    return pl.pallas_call(
        body,
        grid=(PAD // br,),
        in_specs=[
            pl.BlockSpec((2, br, k), lambda i: (0, i, 0)),
            pl.BlockSpec((br, 1), lambda i: (i, 0)),
            pl.BlockSpec((k, ko), lambda i: (0, 0)),
            pl.BlockSpec((1, ko), lambda i: (0, 0)),
            pl.BlockSpec((br, 1), lambda i: (i, 0)),
        ],
        out_specs=pl.BlockSpec((br, ko), lambda i: (i, 0)),
        out_shape=jax.ShapeDtypeStruct((PAD, ko), jnp.float32),
    )(parts, scale_in.reshape(-1, 1), w, b.reshape(1, -1), scale_out.reshape(-1, 1))


def _tc_final(parts, scale):
    br = 1024
    k = parts.shape[2]

    def body(p_ref, s_ref, o_ref):
        o_ref[...] = (p_ref[0] + p_ref[1]) * s_ref[...]

    return pl.pallas_call(
        body,
        grid=(PAD // br,),
        in_specs=[
            pl.BlockSpec((2, br, k), lambda i: (0, i, 0)),
            pl.BlockSpec((br, 1), lambda i: (i, 0)),
        ],
        out_specs=pl.BlockSpec((br, k), lambda i: (i, 0)),
        out_shape=jax.ShapeDtypeStruct((PAD, k), jnp.float32),
    )(parts, scale.reshape(-1, 1))


def kernel(x, edge_index, D_v_beta, D_e_beta_inv, D_e_alpha, D_v_alpha_inv,
           W_v2e_1, b_v2e_1, W_e2v_1, b_e2v_1, W_v2e_2, b_v2e_2, W_e2v_2, b_e2v_2):
    node = edge_index[0].reshape(NW, REAL_PER_TILE)
    hedge = edge_index[1].reshape(NW, REAL_PER_TILE)
    dummy_src = jnp.zeros((NW, DUM_PER_TILE), jnp.int32)
    rot = (jnp.arange(DUM_PER_TILE, dtype=jnp.int32)[None, :]
           + 113 * jnp.arange(NW, dtype=jnp.int32)[:, None]) % DUM_PER_TILE
    dummy_dst = N_NODES + rot
    shp = (NW, 2, NCH_H, CHUNK)
    node_src = jnp.concatenate([node, dummy_src], 1).reshape(shp)
    node_dst = jnp.concatenate([node, dummy_dst], 1).reshape(shp)
    hedge_src = jnp.concatenate([hedge, dummy_src], 1).reshape(shp)
    hedge_dst = jnp.concatenate([hedge, dummy_dst], 1).reshape(shp)

    def padrow(v):
        return jnp.pad(v, (0, PAD - v.shape[0]))

    dvb_p = padrow(D_v_beta)
    debi_p = padrow(D_e_beta_inv)
    dea_p = padrow(D_e_alpha)
    dvai_p = padrow(D_v_alpha_inv)

    seg128 = _make_segsum(128)

    h1 = _tc_first(x, W_v2e_1, b_v2e_1, D_v_beta)
    aggE1 = seg128(h1, node_src, hedge_dst)
    o1 = _tc_mid(aggE1, debi_p, W_e2v_1, b_e2v_1, dea_p)
    aggN1 = seg128(o1, hedge_src, node_dst)
    h2 = _tc_mid(aggN1, dvai_p, W_v2e_2, b_v2e_2, dvb_p)
    aggE2 = seg128(h2, node_src, hedge_dst)
    w2p = jnp.pad(W_e2v_2, ((0, 0), (0, 128 - 40)))
    b2p = jnp.pad(b_e2v_2, (0, 128 - 40))
    o2 = _tc_mid(aggE2, debi_p, w2p, b2p, dea_p)
    aggN2 = seg128(o2, hedge_src, node_dst)
    out = _tc_final(aggN2, dvai_p)
    return out[:N_NODES, :40]


# speedup vs baseline: 1.3442x; 1.3442x over previous
import functools

import jax
import jax.numpy as jnp
from jax import lax
from jax.experimental import pallas as pl
from jax.experimental.pallas import tpu as pltpu
from jax.experimental.pallas import tpu_sc as plsc

N_NODES = 10000
N_HEDGES = 10000
NNZ = 320000
PAD = 10240
NW = 32
CHUNK = 128
NCH = 79
REAL_PER_TILE = NNZ // NW
DUM_PER_TILE = NCH * CHUNK - REAL_PER_TILE
ROWS_PER_SUB = PAD // 16


def _make_segsum(F):
    mesh = plsc.VectorSubcoreMesh(core_axis_name="c", subcore_axis_name="s")

    @functools.partial(
        pl.kernel,
        mesh=mesh,
        out_type=jax.ShapeDtypeStruct((2, PAD, F), jnp.float32),
        scratch_types=[
            pltpu.VMEM((NCH, CHUNK), jnp.int32),
            pltpu.VMEM((NCH, CHUNK), jnp.int32),
            pltpu.VMEM((CHUNK, F), jnp.float32),
            pltpu.VMEM_SHARED((PAD, F), jnp.float32),
            pltpu.SemaphoreType.DMA,
        ],
    )
    def seg(table_hbm, src_hbm, dst_hbm, out_hbm, sidx_v, didx_v, rows0_v, acc_sh, sem):
        c = lax.axis_index("c")
        s = lax.axis_index("s")
        wid = s * 2 + c

        zvec = jnp.zeros((16,), jnp.float32)

        def _zrow(i, carry):
            for j in range(F // 16):
                rows0_v[i, pl.ds(j * 16, 16)] = zvec
            return carry

        lax.fori_loop(0, CHUNK, _zrow, 0)
        for k in range(ROWS_PER_SUB // CHUNK):
            pltpu.sync_copy(rows0_v,
                            acc_sh.at[pl.ds(s * ROWS_PER_SUB + k * CHUNK, CHUNK)])
        plsc.subcore_barrier()

        pltpu.sync_copy(src_hbm.at[wid], sidx_v)
        pltpu.sync_copy(dst_hbm.at[wid], didx_v)

        def _chunk(j, carry):
            pltpu.async_copy(table_hbm.at[sidx_v.at[j]], rows0_v, sem).wait()
            pltpu.sync_copy(rows0_v, acc_sh.at[didx_v.at[j]], add=True)
            return carry

        lax.fori_loop(0, NCH, _chunk, 0)

        plsc.subcore_barrier()
        pltpu.sync_copy(
            acc_sh.at[pl.ds(s * ROWS_PER_SUB, ROWS_PER_SUB)],
            out_hbm.at[c, pl.ds(s * ROWS_PER_SUB, ROWS_PER_SUB)],
        )

    return seg


def _tc_first(x, w, b, scale_out):
    n = x.shape[0]
    br = 1000

    def body(x_ref, w_ref, b_ref, so_ref, o_ref):
        h = jnp.dot(x_ref[...], w_ref[...], preferred_element_type=jnp.float32)
        o_ref[...] = (h + b_ref[...]) * so_ref[...]

    return pl.pallas_call(
        body,
        grid=(n // br,),
        in_specs=[
            pl.BlockSpec((br, 128), lambda i: (i, 0)),
            pl.BlockSpec((128, 128), lambda i: (0, 0)),
            pl.BlockSpec((1, 128), lambda i: (0, 0)),
            pl.BlockSpec((br, 1), lambda i: (i, 0)),
        ],
        out_specs=pl.BlockSpec((br, 128), lambda i: (i, 0)),
        out_shape=jax.ShapeDtypeStruct((n, 128), jnp.float32),
    )(x, w, b.reshape(1, -1), scale_out.reshape(-1, 1))


def _tc_mid(parts, scale_in, w, b, scale_out):
    br = 1024
    k = parts.shape[2]
    ko = w.shape[1]

    def body(p_ref, si_ref, w_ref, b_ref, so_ref, o_ref):
        a = p_ref[0] + p_ref[1]
        t = jnp.maximum(a * si_ref[...], 0.0)
        h = jnp.dot(t, w_ref[...], preferred_element_type=jnp.float32)
        o_ref[...] = (h + b_ref[...]) * so_ref[...]

    return pl.pallas_call(
        body,
        grid=(PAD // br,),
        in_specs=[
            pl.BlockSpec((2, br, k), lambda i: (0, i, 0)),
            pl.BlockSpec((br, 1), lambda i: (i, 0)),
            pl.BlockSpec((k, ko), lambda i: (0, 0)),
            pl.BlockSpec((1, ko), lambda i: (0, 0)),
            pl.BlockSpec((br, 1), lambda i: (i, 0)),
        ],
        out_specs=pl.BlockSpec((br, ko), lambda i: (i, 0)),
        out_shape=jax.ShapeDtypeStruct((PAD, ko), jnp.float32),
    )(parts, scale_in.reshape(-1, 1), w, b.reshape(1, -1), scale_out.reshape(-1, 1))


def _tc_final(parts, scale):
    br = 1024
    k = parts.shape[2]

    def body(p_ref, s_ref, o_ref):
        o_ref[...] = (p_ref[0] + p_ref[1]) * s_ref[...]

    return pl.pallas_call(
        body,
        grid=(PAD // br,),
        in_specs=[
            pl.BlockSpec((2, br, k), lambda i: (0, i, 0)),
            pl.BlockSpec((br, 1), lambda i: (i, 0)),
        ],
        out_specs=pl.BlockSpec((br, k), lambda i: (i, 0)),
        out_shape=jax.ShapeDtypeStruct((PAD, k), jnp.float32),
    )(parts, scale.reshape(-1, 1))


def kernel(x, edge_index, D_v_beta, D_e_beta_inv, D_e_alpha, D_v_alpha_inv,
           W_v2e_1, b_v2e_1, W_e2v_1, b_e2v_1, W_v2e_2, b_v2e_2, W_e2v_2, b_e2v_2):
    node = edge_index[0].reshape(NW, REAL_PER_TILE)
    hedge = edge_index[1].reshape(NW, REAL_PER_TILE)
    dummy_src = jnp.zeros((NW, DUM_PER_TILE), jnp.int32)
    rot = (jnp.arange(DUM_PER_TILE, dtype=jnp.int32)[None, :]
           + 113 * jnp.arange(NW, dtype=jnp.int32)[:, None]) % DUM_PER_TILE
    dummy_dst = N_NODES + rot
    shp = (NW, NCH, CHUNK)
    node_src = jnp.concatenate([node, dummy_src], 1).reshape(shp)
    node_dst = jnp.concatenate([node, dummy_dst], 1).reshape(shp)
    hedge_src = jnp.concatenate([hedge, dummy_src], 1).reshape(shp)
    hedge_dst = jnp.concatenate([hedge, dummy_dst], 1).reshape(shp)

    def padrow(v):
        return jnp.pad(v, (0, PAD - v.shape[0]))

    dvb_p = padrow(D_v_beta)
    debi_p = padrow(D_e_beta_inv)
    dea_p = padrow(D_e_alpha)
    dvai_p = padrow(D_v_alpha_inv)

    seg128 = _make_segsum(128)

    h1 = _tc_first(x, W_v2e_1, b_v2e_1, D_v_beta)
    aggE1 = seg128(h1, node_src, hedge_dst)
    o1 = _tc_mid(aggE1, debi_p, W_e2v_1, b_e2v_1, dea_p)
    aggN1 = seg128(o1, hedge_src, node_dst)
    h2 = _tc_mid(aggN1, dvai_p, W_v2e_2, b_v2e_2, dvb_p)
    aggE2 = seg128(h2, node_src, hedge_dst)
    w2p = jnp.pad(W_e2v_2, ((0, 0), (0, 128 - 40)))
    b2p = jnp.pad(b_e2v_2, (0, 128 - 40))
    o2 = _tc_mid(aggE2, debi_p, w2p, b2p, dea_p)
    aggN2 = seg128(o2, hedge_src, node_dst)
    out = _tc_final(aggN2, dvai_p)
    return out[:N_NODES, :40]
